# baseline (speedup 1.0000x reference)
import math

import jax
import jax.numpy as jnp
from jax import lax
from jax.experimental import pallas as pl
from jax.experimental.pallas import tpu as pltpu
from jax.experimental.pallas import tpu_sc as plsc

_NUM_PIDS = 100000
_NUM_CQ = 5000
_D = 64
_B = 1024
_SCALAR = 30.0
_MARGIN = 0.3

_NC = 2
_NS = 16
_NW = _NC * _NS
_BPW = _B // _NW

_BN = 2000
_NBLK = _NUM_PIDS // _BN

_RB = 256
_NRB = (2 * _B) // _RB


def _sc_gather_body(roi_hbm, t1_hbm, t2_hbm,
                    g1_out, g2_out,
                    idx_v, buf_1, buf_2, sem_1, sem_2):
    wid = lax.axis_index("s") * _NC + lax.axis_index("c")
    base = wid * _BPW
    pltpu.sync_copy(roi_hbm.at[pl.ds(base, _BPW)], idx_v)
    for c in range(_BPW // 16):
        sl = pl.ds(c * 16, 16)
        idx_v[sl] = jnp.maximum(idx_v[sl] - 1, 0)
    c1 = pltpu.async_copy(t1_hbm.at[idx_v], buf_1, sem_1)
    c2 = pltpu.async_copy(t2_hbm.at[idx_v], buf_2, sem_2)
    c1.wait()
    pltpu.sync_copy(buf_1, g1_out.at[pl.ds(base, _BPW)])
    c2.wait()
    pltpu.sync_copy(buf_2, g2_out.at[pl.ds(base, _BPW)])


def _sc_gather(roi_label, t1, t2):
    mesh = plsc.VectorSubcoreMesh(core_axis_name="c", subcore_axis_name="s")
    f32, i32 = jnp.float32, jnp.int32
    k = pl.kernel(
        _sc_gather_body,
        out_type=[
            jax.ShapeDtypeStruct((_B, 2 * _D), f32),
            jax.ShapeDtypeStruct((_B, 2 * _D), f32),
        ],
        mesh=mesh,
        scratch_types=[
            pltpu.VMEM((_BPW,), i32),
            pltpu.VMEM((_BPW, 2 * _D), f32),
            pltpu.VMEM((_BPW, 2 * _D), f32),
            pltpu.SemaphoreType.DMA,
            pltpu.SemaphoreType.DMA,
        ],
        compiler_params=pltpu.CompilerParams(use_tc_tiling_on_sc=True,
                                             needs_layout_passes=False),
    )
    return k(roi_label, t1, t2)


def _lse_body(x_ref, lut_ref, cq_ref, o_ref, m_s, s_s):
    i = pl.program_id(0)

    @pl.when(i == 0)
    def _init():
        m_s[...] = jnp.full((_B, 1), -1e30, jnp.float32)
        s_s[...] = jnp.zeros((_B, 1), jnp.float32)

    def update(w):
        n = w.shape[0]
        logits = lax.dot_general(
            x_ref[...], w, (((1,), (1,)), ((), ())),
            preferred_element_type=jnp.float32
        ).astype(jnp.bfloat16)
        bm = jnp.max(logits, axis=1, keepdims=True).astype(jnp.float32)
        m_old = m_s[...]
        m_new = jnp.maximum(m_old, bm)
        p = jnp.exp(logits - m_new.astype(jnp.bfloat16))
        bs = lax.dot_general(
            p, jnp.ones((n, 1), jnp.bfloat16), (((1,), (0,)), ((), ())),
            preferred_element_type=jnp.float32)
        s_s[...] = s_s[...] * jnp.exp(m_old - m_new) + bs
        m_s[...] = m_new

    @pl.when(i < _NBLK)
    def _lut():
        update(lut_ref[...])

    @pl.when(i == _NBLK)
    def _fin():
        update(cq_ref[...])
        o_ref[...] = m_s[...] + jnp.log(s_s[...])


def _stream_lse(x_bf, lut_bf, cq_bf):
    return pl.pallas_call(
        _lse_body,
        grid=(_NBLK + 1,),
        in_specs=[
            pl.BlockSpec((_B, _D), lambda i: (0, 0)),
            pl.BlockSpec((_BN, _D), lambda i: (jnp.minimum(i, _NBLK - 1), 0)),
            pl.BlockSpec((_NUM_CQ, _D), lambda i: (0, 0)),
        ],
        out_specs=pl.BlockSpec((_B, 1), lambda i: (0, 0)),
        out_shape=jax.ShapeDtypeStruct((_B, 1), jnp.float32),
        scratch_shapes=[
            pltpu.VMEM((_B, 1), jnp.float32),
            pltpu.VMEM((_B, 1), jnp.float32),
        ],
    )(x_bf, lut_bf, cq_bf)


def _combine_body(x_ref, first_ref, second_ref, mid_ref, fidx_ref,
                  labc_ref, lab2c_ref, lab2r_ref, w_ref, b_ref,
                  lse_ref, tgt_ref,
                  oim_ref, trip_ref, cov_ref,
                  feat_s, sqr_s, accl_s, accn_s):
    i = pl.program_id(0)

    @pl.when(i == 0)
    def _setup():
        lab = labc_ref[...]
        keep = lab >= 0
        wid = keep & (lab != 20000)
        bfh = wid & (fidx_ref[...] <= mid_ref[...])
        proto = jnp.where(bfh, first_ref[...], second_ref[...])
        feat_s[0:_B, :] = x_ref[...]
        feat_s[_B:2 * _B, :] = proto
        f = feat_s[...]
        sqr_s[...] = lax.dot_general(
            jnp.ones((1, _D), jnp.float32), f * f, (((1,), (1,)), ((), ())),
            preferred_element_type=jnp.float32)
        accl_s[0, 0] = 0.0
        accn_s[0, 0] = 0.0

        tdot = jnp.sum(x_ref[...] * tgt_ref[...], axis=1,
                       keepdims=True) * _SCALAR
        nll = jnp.where(wid, lse_ref[...] - tdot, 0.0)
        cnt = jnp.sum(wid.astype(jnp.float32), keepdims=True)
        oim_ref[...] = jnp.sum(nll, keepdims=True) / jnp.maximum(cnt, 1.0)

        xw = x_ref[...] * w_ref[...] + b_ref[...]
        mn = jnp.min(xw, axis=1, keepdims=True)
        xw = xw - mn
        mx = jnp.max(xw, axis=1, keepdims=True)
        xw = xw / mx * (math.pi / 2.0)
        z = math.sqrt(2.0) * (jnp.cos(xw) + jnp.sin(xw))
        zm = jnp.where(keep, z, 0.0)
        nk = jnp.sum(keep.astype(jnp.float32))
        c = lax.dot_general(zm, zm, (((0,), (0,)), ((), ())),
                            preferred_element_type=jnp.float32) / nk
        e = jnp.sum(zm, axis=0, keepdims=True) / nk
        outer = lax.dot_general(e, e, (((0,), (0,)), ((), ())),
                                preferred_element_type=jnp.float32)
        cm = c - outer
        cm2 = cm * cm
        di = (lax.broadcasted_iota(jnp.int32, (_D, _D), 0)
              == lax.broadcasted_iota(jnp.int32, (_D, _D), 1))
        cov_ref[...] = jnp.sum(jnp.where(di, 0.0, cm2), keepdims=True)

    r0 = i * _RB
    fb = feat_s[pl.ds(r0, _RB), :]
    g = lax.dot_general(fb, feat_s[...], (((1,), (1,)), ((), ())),
                        preferred_element_type=jnp.float32)
    sqc = jnp.sum(fb * fb, axis=1, keepdims=True)
    d2 = sqc + sqr_s[...] - 2.0 * g
    dist = jnp.sqrt(jnp.clip(d2, 1e-12, None))
    labc = lab2c_ref[...]
    labr = lab2r_ref[...]
    same = labc == labr
    rvec = lax.broadcasted_iota(jnp.int32, (_RB, 1), 0) + r0
    cvec = lax.broadcasted_iota(jnp.int32, (1, 2 * _B), 1)
    eye = rvec == cvec
    vc = (labc >= 0) & ((rvec < _B) | (labc != 20000))
    vr = (labr >= 0) & ((cvec < _B) | (labr != 20000))
    pair_ok = vc & vr
    pos = same & (~eye) & pair_ok
    neg = (~same) & pair_ok
    hp = jnp.max(jnp.where(pos, dist, -1e9), axis=1, keepdims=True)
    hn = jnp.min(jnp.where(neg, dist, 1e9), axis=1, keepdims=True)
    anyp = jnp.any(pos, axis=1, keepdims=True)
    anyn = jnp.any(neg, axis=1, keepdims=True)
    validr = anyp & anyn
    l = jnp.maximum(hp - hn + _MARGIN, 0.0)
    l = jnp.where(validr, l, 0.0)
    accl_s[0, 0] += jnp.sum(l)
    accn_s[0, 0] += jnp.sum(validr.astype(jnp.float32))

    @pl.when(i == _NRB - 1)
    def _fin():
        val = accl_s[0, 0] / jnp.maximum(accn_s[0, 0], 1.0)
        trip_ref[...] = jnp.full((1, 1), val, jnp.float32)


def _combine_losses(x, first_rows, second_rows, mid_col, fidx_col,
                    lab_col, lab2_col, lab2_row, rff_w, b_row,
                    lse, tgt_rows):
    full = lambda s: pl.BlockSpec(s, lambda i, n=len(s): (0,) * n)
    return pl.pallas_call(
        _combine_body,
        grid=(_NRB,),
        in_specs=[
            full((_B, _D)),
            full((_B, _D)),
            full((_B, _D)),
            full((_B, 1)),
            full((_B, 1)),
            full((_B, 1)),
            pl.BlockSpec((_RB, 1), lambda i: (i, 0)),
            full((1, 2 * _B)),
            full((1, 1)),
            full((1, _D)),
            full((_B, 1)),
            full((_B, _D)),
        ],
        out_specs=[full((1, 1)), full((1, 1)), full((1, 1))],
        out_shape=[
            jax.ShapeDtypeStruct((1, 1), jnp.float32),
            jax.ShapeDtypeStruct((1, 1), jnp.float32),
            jax.ShapeDtypeStruct((1, 1), jnp.float32),
        ],
        scratch_shapes=[
            pltpu.VMEM((2 * _B, _D), jnp.float32),
            pltpu.VMEM((1, 2 * _B), jnp.float32),
            pltpu.SMEM((1, 1), jnp.float32),
            pltpu.SMEM((1, 1), jnp.float32),
        ],
    )(x, first_rows, second_rows, mid_col, fidx_col,
      lab_col, lab2_col, lab2_row, rff_w, b_row, lse, tgt_rows)


def kernel(inputs, roi_label, frame_idxes, lut, cq, first_pos_sample,
           second_pos_sample, middle_of_frame_idx, rff_w, rff_b):
    label_all = roi_label - 1
    fidx = jnp.repeat(frame_idxes, _B // frame_idxes.shape[0])

    x_bf = (inputs * _SCALAR).astype(jnp.bfloat16)
    lut_bf = lut.astype(jnp.bfloat16)
    cq_bf = cq.astype(jnp.bfloat16)
    lse = _stream_lse(x_bf, lut_bf, cq_bf)

    mid_f = lax.bitcast_convert_type(middle_of_frame_idx, jnp.float32)
    t1 = jnp.concatenate([lut, first_pos_sample], axis=1)
    t2 = jnp.concatenate([second_pos_sample, mid_f.reshape(-1, 1),
                          jnp.zeros((_NUM_PIDS, _D - 1), jnp.float32)],
                         axis=1)
    g1, g2 = _sc_gather(roi_label, t1, t2)
    tgt_rows = g1[:, :_D]
    first_rows = g1[:, _D:]
    second_rows = g2[:, :_D]
    mid_col = lax.bitcast_convert_type(g2[:, _D], jnp.int32).reshape(_B, 1)

    lab_col = label_all.reshape(_B, 1)
    lab2 = jnp.concatenate([label_all, label_all])
    lab2_col = lab2.reshape(2 * _B, 1)
    lab2_row = lab2.reshape(1, 2 * _B)
    fidx_col = fidx.reshape(_B, 1)
    b_row = rff_b.reshape(1, _D)

    oim, trip, cov = _combine_losses(
        inputs, first_rows, second_rows, mid_col, fidx_col,
        lab_col, lab2_col, lab2_row, rff_w, b_row, lse, tgt_rows)
    return jnp.stack([oim[0, 0], cov[0, 0], trip[0, 0]])


# speedup vs baseline: 1.2865x; 1.2865x over previous
import math

import jax
import jax.numpy as jnp
from jax import lax
from jax.experimental import pallas as pl
from jax.experimental.pallas import tpu as pltpu
from jax.experimental.pallas import tpu_sc as plsc

_NUM_PIDS = 100000
_NUM_CQ = 5000
_D = 64
_B = 1024
_SCALAR = 30.0
_MARGIN = 0.3

_NC = 2
_NS = 16
_NW = _NC * _NS
_BPW = _B // _NW

_BN = 2000
_NBLK = _NUM_PIDS // _BN

_RB = 256
_NRB = (2 * _B) // _RB


def _sc_gather_body(roi_hbm, lut_hbm, first_hbm, second_hbm, mid_hbm,
                    lutt_out, firstt_out, secondt_out, mid_out,
                    idx_v, tidx_v, lane_v, mal_v,
                    buf_l, buf_f, buf_s, midbuf, mid_v,
                    sem_l, sem_f, sem_s, sem_m):
    wid = lax.axis_index("s") * _NC + lax.axis_index("c")
    base = wid * _BPW
    pltpu.sync_copy(roi_hbm.at[pl.ds(base, _BPW)], idx_v)
    for c in range(_BPW // 16):
        sl = pl.ds(c * 16, 16)
        v = jnp.maximum(idx_v[sl] - 1, 0)
        idx_v[sl] = v
        tidx_v[sl] = v >> 3
        lane_v[sl] = v & 15
        mal_v[sl] = (v >> 4) << 4
    waits = []
    for c in range(_BPW // 16):
        tch = tidx_v[pl.ds(c * 16, 16)]
        ach = mal_v[pl.ds(c * 16, 16)]
        for jj in range(16):
            j = c * 16 + jj
            t = tch[jj]
            waits.append(pltpu.async_copy(
                lut_hbm.at[pl.ds(t, 1)], buf_l.at[pl.ds(j, 1)], sem_l))
            waits.append(pltpu.async_copy(
                first_hbm.at[pl.ds(t, 1)], buf_f.at[pl.ds(j, 1)], sem_f))
            waits.append(pltpu.async_copy(
                second_hbm.at[pl.ds(t, 1)], buf_s.at[pl.ds(j, 1)], sem_s))
            a = pl.multiple_of(ach[jj], 16)
            waits.append(pltpu.async_copy(
                mid_hbm.at[pl.ds(a, 16)], midbuf.at[j], sem_m))
    for w in waits:
        w.wait()
    for c in range(_BPW // 16):
        sl = pl.ds(c * 16, 16)
        d0 = lax.iota(jnp.int32, 16) + (16 * c)
        mid_v[sl] = plsc.load_gather(midbuf, [d0, lane_v[sl]])
    pltpu.sync_copy(buf_l, lutt_out.at[pl.ds(base, _BPW)])
    pltpu.sync_copy(buf_f, firstt_out.at[pl.ds(base, _BPW)])
    pltpu.sync_copy(buf_s, secondt_out.at[pl.ds(base, _BPW)])
    pltpu.sync_copy(mid_v, mid_out.at[pl.ds(base, _BPW)])


def _sc_gather(roi_label, lut3, first3, second3, mid1d):
    mesh = plsc.VectorSubcoreMesh(core_axis_name="c", subcore_axis_name="s")
    f32, i32 = jnp.float32, jnp.int32
    k = pl.kernel(
        _sc_gather_body,
        out_type=[
            jax.ShapeDtypeStruct((_B, 8, _D), f32),
            jax.ShapeDtypeStruct((_B, 8, _D), f32),
            jax.ShapeDtypeStruct((_B, 8, _D), f32),
            jax.ShapeDtypeStruct((_B,), i32),
        ],
        mesh=mesh,
        scratch_types=[
            pltpu.VMEM((_BPW,), i32),
            pltpu.VMEM((_BPW,), i32),
            pltpu.VMEM((_BPW,), i32),
            pltpu.VMEM((_BPW,), i32),
            pltpu.VMEM((_BPW, 8, _D), f32),
            pltpu.VMEM((_BPW, 8, _D), f32),
            pltpu.VMEM((_BPW, 8, _D), f32),
            pltpu.VMEM((_BPW, 16), i32),
            pltpu.VMEM((_BPW,), i32),
            pltpu.SemaphoreType.DMA,
            pltpu.SemaphoreType.DMA,
            pltpu.SemaphoreType.DMA,
            pltpu.SemaphoreType.DMA,
        ],
        compiler_params=pltpu.CompilerParams(use_tc_tiling_on_sc=True,
                                             needs_layout_passes=False),
    )
    return k(roi_label, lut3, first3, second3, mid1d)


def _lse_body(x_ref, lut_ref, cq_ref, o_ref, m_s, s_s):
    i = pl.program_id(0)

    @pl.when(i == 0)
    def _init():
        m_s[...] = jnp.full((_B, 1), -1e30, jnp.float32)
        s_s[...] = jnp.zeros((_B, 1), jnp.float32)

    def update(w):
        n = w.shape[0]
        logits = lax.dot_general(
            x_ref[...], w, (((1,), (1,)), ((), ())),
            preferred_element_type=jnp.float32
        ).astype(jnp.bfloat16)
        bm = jnp.max(logits, axis=1, keepdims=True).astype(jnp.float32)
        m_old = m_s[...]
        m_new = jnp.maximum(m_old, bm)
        p = jnp.exp(logits - m_new.astype(jnp.bfloat16))
        bs = lax.dot_general(
            p, jnp.ones((n, 1), jnp.bfloat16), (((1,), (0,)), ((), ())),
            preferred_element_type=jnp.float32)
        s_s[...] = s_s[...] * jnp.exp(m_old - m_new) + bs
        m_s[...] = m_new

    @pl.when(i < _NBLK)
    def _lut():
        update(lut_ref[...])

    @pl.when(i == _NBLK)
    def _fin():
        update(cq_ref[...])
        o_ref[...] = m_s[...] + jnp.log(s_s[...])


def _stream_lse(x_bf, lut_bf, cq_bf):
    return pl.pallas_call(
        _lse_body,
        grid=(_NBLK + 1,),
        in_specs=[
            pl.BlockSpec((_B, _D), lambda i: (0, 0)),
            pl.BlockSpec((_BN, _D), lambda i: (jnp.minimum(i, _NBLK - 1), 0)),
            pl.BlockSpec((_NUM_CQ, _D), lambda i: (0, 0)),
        ],
        out_specs=pl.BlockSpec((_B, 1), lambda i: (0, 0)),
        out_shape=jax.ShapeDtypeStruct((_B, 1), jnp.float32),
        scratch_shapes=[
            pltpu.VMEM((_B, 1), jnp.float32),
            pltpu.VMEM((_B, 1), jnp.float32),
        ],
    )(x_bf, lut_bf, cq_bf)


def _combine_body(x_ref, first_ref, second_ref, mid_ref, fidx_ref,
                  labc_ref, lab2c_ref, lab2r_ref, w_ref, b_ref,
                  lse_ref, tgt_ref,
                  oim_ref, trip_ref, cov_ref,
                  feat_s, sqr_s, accl_s, accn_s):
    i = pl.program_id(0)

    @pl.when(i == 0)
    def _setup():
        lab = labc_ref[...]
        keep = lab >= 0
        wid = keep & (lab != 20000)
        bfh = wid & (fidx_ref[...] <= mid_ref[...])

        rj = jnp.maximum(lab, 0) & 7
        def extract(t3_ref):
            acc = jnp.where(rj == 0, t3_ref[:, 0, :], 0.0)
            for r in range(1, 8):
                acc = acc + jnp.where(rj == r, t3_ref[:, r, :], 0.0)
            return acc

        proto = jnp.where(bfh, extract(first_ref), extract(second_ref))
        feat_s[0:_B, :] = x_ref[...]
        feat_s[_B:2 * _B, :] = proto
        f = feat_s[...]
        sqr_s[...] = lax.dot_general(
            jnp.ones((1, _D), jnp.float32), f * f, (((1,), (1,)), ((), ())),
            preferred_element_type=jnp.float32)
        accl_s[0, 0] = 0.0
        accn_s[0, 0] = 0.0

        tdot = jnp.sum(x_ref[...] * extract(tgt_ref), axis=1,
                       keepdims=True) * _SCALAR
        nll = jnp.where(wid, lse_ref[...] - tdot, 0.0)
        cnt = jnp.sum(wid.astype(jnp.float32), keepdims=True)
        oim_ref[...] = jnp.sum(nll, keepdims=True) / jnp.maximum(cnt, 1.0)

        xw = x_ref[...] * w_ref[...] + b_ref[...]
        mn = jnp.min(xw, axis=1, keepdims=True)
        xw = xw - mn
        mx = jnp.max(xw, axis=1, keepdims=True)
        xw = xw / mx * (math.pi / 2.0)
        z = math.sqrt(2.0) * (jnp.cos(xw) + jnp.sin(xw))
        zm = jnp.where(keep, z, 0.0)
        nk = jnp.sum(keep.astype(jnp.float32))
        c = lax.dot_general(zm, zm, (((0,), (0,)), ((), ())),
                            preferred_element_type=jnp.float32) / nk
        e = jnp.sum(zm, axis=0, keepdims=True) / nk
        outer = lax.dot_general(e, e, (((0,), (0,)), ((), ())),
                                preferred_element_type=jnp.float32)
        cm = c - outer
        cm2 = cm * cm
        di = (lax.broadcasted_iota(jnp.int32, (_D, _D), 0)
              == lax.broadcasted_iota(jnp.int32, (_D, _D), 1))
        cov_ref[...] = jnp.sum(jnp.where(di, 0.0, cm2), keepdims=True)

    r0 = i * _RB
    fb = feat_s[pl.ds(r0, _RB), :]
    g = lax.dot_general(fb, feat_s[...], (((1,), (1,)), ((), ())),
                        preferred_element_type=jnp.float32)
    sqc = jnp.sum(fb * fb, axis=1, keepdims=True)
    d2 = sqc + sqr_s[...] - 2.0 * g
    dist = jnp.sqrt(jnp.clip(d2, 1e-12, None))
    labc = lab2c_ref[...]
    labr = lab2r_ref[...]
    same = labc == labr
    rvec = lax.broadcasted_iota(jnp.int32, (_RB, 1), 0) + r0
    cvec = lax.broadcasted_iota(jnp.int32, (1, 2 * _B), 1)
    eye = rvec == cvec
    vc = (labc >= 0) & ((rvec < _B) | (labc != 20000))
    vr = (labr >= 0) & ((cvec < _B) | (labr != 20000))
    pair_ok = vc & vr
    pos = same & (~eye) & pair_ok
    neg = (~same) & pair_ok
    hp = jnp.max(jnp.where(pos, dist, -1e9), axis=1, keepdims=True)
    hn = jnp.min(jnp.where(neg, dist, 1e9), axis=1, keepdims=True)
    anyp = jnp.any(pos, axis=1, keepdims=True)
    anyn = jnp.any(neg, axis=1, keepdims=True)
    validr = anyp & anyn
    l = jnp.maximum(hp - hn + _MARGIN, 0.0)
    l = jnp.where(validr, l, 0.0)
    accl_s[0, 0] += jnp.sum(l)
    accn_s[0, 0] += jnp.sum(validr.astype(jnp.float32))

    @pl.when(i == _NRB - 1)
    def _fin():
        val = accl_s[0, 0] / jnp.maximum(accn_s[0, 0], 1.0)
        trip_ref[...] = jnp.full((1, 1), val, jnp.float32)


def _combine_losses(x, first_rows, second_rows, mid_col, fidx_col,
                    lab_col, lab2_col, lab2_row, rff_w, b_row,
                    lse, tgt_rows):
    full = lambda s: pl.BlockSpec(s, lambda i, n=len(s): (0,) * n)
    return pl.pallas_call(
        _combine_body,
        grid=(_NRB,),
        in_specs=[
            full((_B, _D)),
            full((_B, 8, _D)),
            full((_B, 8, _D)),
            full((_B, 1)),
            full((_B, 1)),
            full((_B, 1)),
            pl.BlockSpec((_RB, 1), lambda i: (i, 0)),
            full((1, 2 * _B)),
            full((1, 1)),
            full((1, _D)),
            full((_B, 1)),
            full((_B, 8, _D)),
        ],
        out_specs=[full((1, 1)), full((1, 1)), full((1, 1))],
        out_shape=[
            jax.ShapeDtypeStruct((1, 1), jnp.float32),
            jax.ShapeDtypeStruct((1, 1), jnp.float32),
            jax.ShapeDtypeStruct((1, 1), jnp.float32),
        ],
        scratch_shapes=[
            pltpu.VMEM((2 * _B, _D), jnp.float32),
            pltpu.VMEM((1, 2 * _B), jnp.float32),
            pltpu.SMEM((1, 1), jnp.float32),
            pltpu.SMEM((1, 1), jnp.float32),
        ],
    )(x, first_rows, second_rows, mid_col, fidx_col,
      lab_col, lab2_col, lab2_row, rff_w, b_row, lse, tgt_rows)


def kernel(inputs, roi_label, frame_idxes, lut, cq, first_pos_sample,
           second_pos_sample, middle_of_frame_idx, rff_w, rff_b):
    label_all = roi_label - 1
    fidx = jnp.repeat(frame_idxes, _B // frame_idxes.shape[0])

    x_bf = (inputs * _SCALAR).astype(jnp.bfloat16)
    lut_bf = lut.astype(jnp.bfloat16)
    cq_bf = cq.astype(jnp.bfloat16)
    lse = _stream_lse(x_bf, lut_bf, cq_bf)

    lut3 = lut.reshape(_NUM_PIDS // 8, 8, _D)
    first3 = first_pos_sample.reshape(_NUM_PIDS // 8, 8, _D)
    second3 = second_pos_sample.reshape(_NUM_PIDS // 8, 8, _D)
    tgt_rows, first_rows, second_rows, mid = _sc_gather(
        roi_label, lut3, first3, second3, middle_of_frame_idx)

    lab_col = label_all.reshape(_B, 1)
    lab2 = jnp.concatenate([label_all, label_all])
    lab2_col = lab2.reshape(2 * _B, 1)
    lab2_row = lab2.reshape(1, 2 * _B)
    fidx_col = fidx.reshape(_B, 1)
    mid_col = mid.reshape(_B, 1)
    b_row = rff_b.reshape(1, _D)

    oim, trip, cov = _combine_losses(
        inputs, first_rows, second_rows, mid_col, fidx_col,
        lab_col, lab2_col, lab2_row, rff_w, b_row, lse, tgt_rows)
    return jnp.stack([oim[0, 0], cov[0, 0], trip[0, 0]])


# speedup vs baseline: 1.3777x; 1.0709x over previous
import math

import jax
import jax.numpy as jnp
from jax import lax
from jax.experimental import pallas as pl
from jax.experimental.pallas import tpu as pltpu
from jax.experimental.pallas import tpu_sc as plsc

_NUM_PIDS = 100000
_NUM_CQ = 5000
_D = 64
_B = 1024
_SCALAR = 30.0
_MARGIN = 0.3

_NC = 2
_NS = 16
_NW = _NC * _NS
_BPW = _B // _NW

_BN = 4000
_NBLK = _NUM_PIDS // _BN

_RB = 256
_NRB = (2 * _B) // _RB


def _sc_gather_body(roi_hbm, lut_hbm, first_hbm, second_hbm, mid_hbm,
                    lutt_out, firstt_out, secondt_out, mid_out,
                    idx_v, tidx_v, lane_v, mal_v,
                    buf_l, buf_f, buf_s, midbuf, mid_v,
                    sem_l, sem_f, sem_s, sem_m):
    wid = lax.axis_index("s") * _NC + lax.axis_index("c")
    base = wid * _BPW
    pltpu.sync_copy(roi_hbm.at[pl.ds(base, _BPW)], idx_v)
    for c in range(_BPW // 16):
        sl = pl.ds(c * 16, 16)
        v = jnp.maximum(idx_v[sl] - 1, 0)
        idx_v[sl] = v
        tidx_v[sl] = v >> 3
        lane_v[sl] = v & 15
        mal_v[sl] = (v >> 4) << 4
    waits = []
    for c in range(_BPW // 16):
        tch = tidx_v[pl.ds(c * 16, 16)]
        ach = mal_v[pl.ds(c * 16, 16)]
        for jj in range(16):
            j = c * 16 + jj
            t = tch[jj]
            waits.append(pltpu.async_copy(
                lut_hbm.at[pl.ds(t, 1)], buf_l.at[pl.ds(j, 1)], sem_l))
            waits.append(pltpu.async_copy(
                first_hbm.at[pl.ds(t, 1)], buf_f.at[pl.ds(j, 1)], sem_f))
            waits.append(pltpu.async_copy(
                second_hbm.at[pl.ds(t, 1)], buf_s.at[pl.ds(j, 1)], sem_s))
            a = pl.multiple_of(ach[jj], 16)
            waits.append(pltpu.async_copy(
                mid_hbm.at[pl.ds(a, 16)], midbuf.at[j], sem_m))
    for w in waits:
        w.wait()
    for c in range(_BPW // 16):
        sl = pl.ds(c * 16, 16)
        d0 = lax.iota(jnp.int32, 16) + (16 * c)
        mid_v[sl] = plsc.load_gather(midbuf, [d0, lane_v[sl]])
    pltpu.sync_copy(buf_l, lutt_out.at[pl.ds(base, _BPW)])
    pltpu.sync_copy(buf_f, firstt_out.at[pl.ds(base, _BPW)])
    pltpu.sync_copy(buf_s, secondt_out.at[pl.ds(base, _BPW)])
    pltpu.sync_copy(mid_v, mid_out.at[pl.ds(base, _BPW)])


def _sc_gather(roi_label, lut3, first3, second3, mid1d):
    mesh = plsc.VectorSubcoreMesh(core_axis_name="c", subcore_axis_name="s")
    f32, i32 = jnp.float32, jnp.int32
    k = pl.kernel(
        _sc_gather_body,
        out_type=[
            jax.ShapeDtypeStruct((_B, 8, _D), f32),
            jax.ShapeDtypeStruct((_B, 8, _D), f32),
            jax.ShapeDtypeStruct((_B, 8, _D), f32),
            jax.ShapeDtypeStruct((_B,), i32),
        ],
        mesh=mesh,
        scratch_types=[
            pltpu.VMEM((_BPW,), i32),
            pltpu.VMEM((_BPW,), i32),
            pltpu.VMEM((_BPW,), i32),
            pltpu.VMEM((_BPW,), i32),
            pltpu.VMEM((_BPW, 8, _D), f32),
            pltpu.VMEM((_BPW, 8, _D), f32),
            pltpu.VMEM((_BPW, 8, _D), f32),
            pltpu.VMEM((_BPW, 16), i32),
            pltpu.VMEM((_BPW,), i32),
            pltpu.SemaphoreType.DMA,
            pltpu.SemaphoreType.DMA,
            pltpu.SemaphoreType.DMA,
            pltpu.SemaphoreType.DMA,
        ],
        compiler_params=pltpu.CompilerParams(use_tc_tiling_on_sc=True,
                                             needs_layout_passes=False),
    )
    return k(roi_label, lut3, first3, second3, mid1d)


def _lse_body(x_ref, lut_ref, cq_ref, o_ref, m_s, s_s):
    i = pl.program_id(0)

    @pl.when(i == 0)
    def _init():
        m_s[...] = jnp.full((_B, 1), -1e30, jnp.float32)
        s_s[...] = jnp.zeros((_B, 1), jnp.float32)

    def update(w):
        n = w.shape[0]
        logits = lax.dot_general(
            x_ref[...], w, (((1,), (1,)), ((), ())),
            preferred_element_type=jnp.float32
        ).astype(jnp.bfloat16)
        bm = jnp.max(logits, axis=1, keepdims=True).astype(jnp.float32)
        m_old = m_s[...]
        m_new = jnp.maximum(m_old, bm)
        p = jnp.exp(logits - m_new.astype(jnp.bfloat16))
        bs = lax.dot_general(
            p, jnp.ones((n, 1), jnp.bfloat16), (((1,), (0,)), ((), ())),
            preferred_element_type=jnp.float32)
        s_s[...] = s_s[...] * jnp.exp(m_old - m_new) + bs
        m_s[...] = m_new

    @pl.when(i < _NBLK)
    def _lut():
        update(lut_ref[...])

    @pl.when(i == _NBLK)
    def _fin():
        update(cq_ref[...])
        o_ref[...] = m_s[...] + jnp.log(s_s[...])


def _stream_lse(x_bf, lut_bf, cq_bf):
    return pl.pallas_call(
        _lse_body,
        grid=(_NBLK + 1,),
        in_specs=[
            pl.BlockSpec((_B, _D), lambda i: (0, 0)),
            pl.BlockSpec((_BN, _D), lambda i: (jnp.minimum(i, _NBLK - 1), 0)),
            pl.BlockSpec((_NUM_CQ, _D), lambda i: (0, 0)),
        ],
        out_specs=pl.BlockSpec((_B, 1), lambda i: (0, 0)),
        out_shape=jax.ShapeDtypeStruct((_B, 1), jnp.float32),
        scratch_shapes=[
            pltpu.VMEM((_B, 1), jnp.float32),
            pltpu.VMEM((_B, 1), jnp.float32),
        ],
    )(x_bf, lut_bf, cq_bf)


def _combine_body(x_ref, first_ref, second_ref, mid_ref, fidx_ref,
                  labc_ref, lab2c_ref, lab2r_ref, w_ref, b_ref,
                  lse_ref, tgt_ref,
                  oim_ref, trip_ref, cov_ref,
                  feat_s, sqr_s, accl_s, accn_s):
    i = pl.program_id(0)

    @pl.when(i == 0)
    def _setup():
        lab = labc_ref[...]
        keep = lab >= 0
        wid = keep & (lab != 20000)
        bfh = wid & (fidx_ref[...] <= mid_ref[...])

        rj = jnp.maximum(lab, 0) & 7
        def extract(t3_ref):
            acc = jnp.where(rj == 0, t3_ref[:, 0, :], 0.0)
            for r in range(1, 8):
                acc = acc + jnp.where(rj == r, t3_ref[:, r, :], 0.0)
            return acc

        proto = jnp.where(bfh, extract(first_ref), extract(second_ref))
        feat_s[0:_B, :] = x_ref[...]
        feat_s[_B:2 * _B, :] = proto
        f = feat_s[...]
        sqr_s[...] = lax.dot_general(
            jnp.ones((1, _D), jnp.float32), f * f, (((1,), (1,)), ((), ())),
            preferred_element_type=jnp.float32)
        accl_s[0, 0] = 0.0
        accn_s[0, 0] = 0.0

        tdot = jnp.sum(x_ref[...] * extract(tgt_ref), axis=1,
                       keepdims=True) * _SCALAR
        nll = jnp.where(wid, lse_ref[...] - tdot, 0.0)
        cnt = jnp.sum(wid.astype(jnp.float32), keepdims=True)
        oim_ref[...] = jnp.sum(nll, keepdims=True) / jnp.maximum(cnt, 1.0)

        xw = x_ref[...] * w_ref[...] + b_ref[...]
        mn = jnp.min(xw, axis=1, keepdims=True)
        xw = xw - mn
        mx = jnp.max(xw, axis=1, keepdims=True)
        xw = xw / mx * (math.pi / 2.0)
        z = math.sqrt(2.0) * (jnp.cos(xw) + jnp.sin(xw))
        zm = jnp.where(keep, z, 0.0)
        nk = jnp.sum(keep.astype(jnp.float32))
        c = lax.dot_general(zm, zm, (((0,), (0,)), ((), ())),
                            preferred_element_type=jnp.float32) / nk
        e = jnp.sum(zm, axis=0, keepdims=True) / nk
        outer = lax.dot_general(e, e, (((0,), (0,)), ((), ())),
                                preferred_element_type=jnp.float32)
        cm = c - outer
        cm2 = cm * cm
        di = (lax.broadcasted_iota(jnp.int32, (_D, _D), 0)
              == lax.broadcasted_iota(jnp.int32, (_D, _D), 1))
        cov_ref[...] = jnp.sum(jnp.where(di, 0.0, cm2), keepdims=True)

    r0 = i * _RB
    fb = feat_s[pl.ds(r0, _RB), :]
    g = lax.dot_general(fb, feat_s[...], (((1,), (1,)), ((), ())),
                        preferred_element_type=jnp.float32)
    sqc = jnp.sum(fb * fb, axis=1, keepdims=True)
    d2 = sqc + sqr_s[...] - 2.0 * g
    dist = jnp.sqrt(jnp.clip(d2, 1e-12, None))
    labc = lab2c_ref[...]
    labr = lab2r_ref[...]
    same = labc == labr
    rvec = lax.broadcasted_iota(jnp.int32, (_RB, 1), 0) + r0
    cvec = lax.broadcasted_iota(jnp.int32, (1, 2 * _B), 1)
    eye = rvec == cvec
    vc = (labc >= 0) & ((rvec < _B) | (labc != 20000))
    vr = (labr >= 0) & ((cvec < _B) | (labr != 20000))
    pair_ok = vc & vr
    pos = same & (~eye) & pair_ok
    neg = (~same) & pair_ok
    hp = jnp.max(jnp.where(pos, dist, -1e9), axis=1, keepdims=True)
    hn = jnp.min(jnp.where(neg, dist, 1e9), axis=1, keepdims=True)
    anyp = jnp.any(pos, axis=1, keepdims=True)
    anyn = jnp.any(neg, axis=1, keepdims=True)
    validr = anyp & anyn
    l = jnp.maximum(hp - hn + _MARGIN, 0.0)
    l = jnp.where(validr, l, 0.0)
    accl_s[0, 0] += jnp.sum(l)
    accn_s[0, 0] += jnp.sum(validr.astype(jnp.float32))

    @pl.when(i == _NRB - 1)
    def _fin():
        val = accl_s[0, 0] / jnp.maximum(accn_s[0, 0], 1.0)
        trip_ref[...] = jnp.full((1, 1), val, jnp.float32)


def _combine_losses(x, first_rows, second_rows, mid_col, fidx_col,
                    lab_col, lab2_col, lab2_row, rff_w, b_row,
                    lse, tgt_rows):
    full = lambda s: pl.BlockSpec(s, lambda i, n=len(s): (0,) * n)
    return pl.pallas_call(
        _combine_body,
        grid=(_NRB,),
        in_specs=[
            full((_B, _D)),
            full((_B, 8, _D)),
            full((_B, 8, _D)),
            full((_B, 1)),
            full((_B, 1)),
            full((_B, 1)),
            pl.BlockSpec((_RB, 1), lambda i: (i, 0)),
            full((1, 2 * _B)),
            full((1, 1)),
            full((1, _D)),
            full((_B, 1)),
            full((_B, 8, _D)),
        ],
        out_specs=[full((1, 1)), full((1, 1)), full((1, 1))],
        out_shape=[
            jax.ShapeDtypeStruct((1, 1), jnp.float32),
            jax.ShapeDtypeStruct((1, 1), jnp.float32),
            jax.ShapeDtypeStruct((1, 1), jnp.float32),
        ],
        scratch_shapes=[
            pltpu.VMEM((2 * _B, _D), jnp.float32),
            pltpu.VMEM((1, 2 * _B), jnp.float32),
            pltpu.SMEM((1, 1), jnp.float32),
            pltpu.SMEM((1, 1), jnp.float32),
        ],
    )(x, first_rows, second_rows, mid_col, fidx_col,
      lab_col, lab2_col, lab2_row, rff_w, b_row, lse, tgt_rows)


def kernel(inputs, roi_label, frame_idxes, lut, cq, first_pos_sample,
           second_pos_sample, middle_of_frame_idx, rff_w, rff_b):
    label_all = roi_label - 1
    fidx = jnp.repeat(frame_idxes, _B // frame_idxes.shape[0])

    x_bf = (inputs * _SCALAR).astype(jnp.bfloat16)
    lut_bf = lut.astype(jnp.bfloat16)
    cq_bf = cq.astype(jnp.bfloat16)
    lse = _stream_lse(x_bf, lut_bf, cq_bf)

    lut3 = lut.reshape(_NUM_PIDS // 8, 8, _D)
    first3 = first_pos_sample.reshape(_NUM_PIDS // 8, 8, _D)
    second3 = second_pos_sample.reshape(_NUM_PIDS // 8, 8, _D)
    tgt_rows, first_rows, second_rows, mid = _sc_gather(
        roi_label, lut3, first3, second3, middle_of_frame_idx)

    lab_col = label_all.reshape(_B, 1)
    lab2 = jnp.concatenate([label_all, label_all])
    lab2_col = lab2.reshape(2 * _B, 1)
    lab2_row = lab2.reshape(1, 2 * _B)
    fidx_col = fidx.reshape(_B, 1)
    mid_col = mid.reshape(_B, 1)
    b_row = rff_b.reshape(1, _D)

    oim, trip, cov = _combine_losses(
        inputs, first_rows, second_rows, mid_col, fidx_col,
        lab_col, lab2_col, lab2_row, rff_w, b_row, lse, tgt_rows)
    return jnp.stack([oim[0, 0], cov[0, 0], trip[0, 0]])


# speedup vs baseline: 1.4029x; 1.0183x over previous
import math

import jax
import jax.numpy as jnp
from jax import lax
from jax.experimental import pallas as pl
from jax.experimental.pallas import tpu as pltpu
from jax.experimental.pallas import tpu_sc as plsc

_NUM_PIDS = 100000
_NUM_CQ = 5000
_D = 64
_B = 1024
_SCALAR = 30.0
_MARGIN = 0.3

_NC = 2
_NS = 16
_NW = _NC * _NS
_BPW = _B // _NW

_BN = 5000
_NBLK = _NUM_PIDS // _BN

_RB = 256
_NRB = (2 * _B) // _RB


def _sc_gather_body(roi_hbm, lut_hbm, first_hbm, second_hbm, mid_hbm,
                    lutt_out, firstt_out, secondt_out, mid_out,
                    idx_v, tidx_v, lane_v, mal_v,
                    buf_l, buf_f, buf_s, midbuf, mid_v,
                    sem_l, sem_f, sem_s, sem_m):
    wid = lax.axis_index("s") * _NC + lax.axis_index("c")
    base = wid * _BPW
    pltpu.sync_copy(roi_hbm.at[pl.ds(base, _BPW)], idx_v)
    for c in range(_BPW // 16):
        sl = pl.ds(c * 16, 16)
        v = jnp.maximum(idx_v[sl] - 1, 0)
        idx_v[sl] = v
        tidx_v[sl] = v >> 3
        lane_v[sl] = v & 15
        mal_v[sl] = (v >> 4) << 4
    waits = []
    for c in range(_BPW // 16):
        tch = tidx_v[pl.ds(c * 16, 16)]
        ach = mal_v[pl.ds(c * 16, 16)]
        for jj in range(16):
            j = c * 16 + jj
            t = tch[jj]
            waits.append(pltpu.async_copy(
                lut_hbm.at[pl.ds(t, 1)], buf_l.at[pl.ds(j, 1)], sem_l))
            waits.append(pltpu.async_copy(
                first_hbm.at[pl.ds(t, 1)], buf_f.at[pl.ds(j, 1)], sem_f))
            waits.append(pltpu.async_copy(
                second_hbm.at[pl.ds(t, 1)], buf_s.at[pl.ds(j, 1)], sem_s))
            a = pl.multiple_of(ach[jj], 16)
            waits.append(pltpu.async_copy(
                mid_hbm.at[pl.ds(a, 16)], midbuf.at[j], sem_m))
    for w in waits:
        w.wait()
    for c in range(_BPW // 16):
        sl = pl.ds(c * 16, 16)
        d0 = lax.iota(jnp.int32, 16) + (16 * c)
        mid_v[sl] = plsc.load_gather(midbuf, [d0, lane_v[sl]])
    pltpu.sync_copy(buf_l, lutt_out.at[pl.ds(base, _BPW)])
    pltpu.sync_copy(buf_f, firstt_out.at[pl.ds(base, _BPW)])
    pltpu.sync_copy(buf_s, secondt_out.at[pl.ds(base, _BPW)])
    pltpu.sync_copy(mid_v, mid_out.at[pl.ds(base, _BPW)])


def _sc_gather(roi_label, lut3, first3, second3, mid1d):
    mesh = plsc.VectorSubcoreMesh(core_axis_name="c", subcore_axis_name="s")
    f32, i32 = jnp.float32, jnp.int32
    k = pl.kernel(
        _sc_gather_body,
        out_type=[
            jax.ShapeDtypeStruct((_B, 8, _D), f32),
            jax.ShapeDtypeStruct((_B, 8, _D), f32),
            jax.ShapeDtypeStruct((_B, 8, _D), f32),
            jax.ShapeDtypeStruct((_B,), i32),
        ],
        mesh=mesh,
        scratch_types=[
            pltpu.VMEM((_BPW,), i32),
            pltpu.VMEM((_BPW,), i32),
            pltpu.VMEM((_BPW,), i32),
            pltpu.VMEM((_BPW,), i32),
            pltpu.VMEM((_BPW, 8, _D), f32),
            pltpu.VMEM((_BPW, 8, _D), f32),
            pltpu.VMEM((_BPW, 8, _D), f32),
            pltpu.VMEM((_BPW, 16), i32),
            pltpu.VMEM((_BPW,), i32),
            pltpu.SemaphoreType.DMA,
            pltpu.SemaphoreType.DMA,
            pltpu.SemaphoreType.DMA,
            pltpu.SemaphoreType.DMA,
        ],
        compiler_params=pltpu.CompilerParams(use_tc_tiling_on_sc=True,
                                             needs_layout_passes=False),
    )
    return k(roi_label, lut3, first3, second3, mid1d)


def _lse_body(x_ref, lut_ref, cq_ref, o_ref, m_s, s_s):
    i = pl.program_id(0)

    @pl.when(i == 0)
    def _init():
        m_s[...] = jnp.full((_B, 1), -1e30, jnp.float32)
        s_s[...] = jnp.zeros((_B, 1), jnp.float32)

    def update(w):
        n = w.shape[0]
        logits = lax.dot_general(
            x_ref[...], w, (((1,), (1,)), ((), ())),
            preferred_element_type=jnp.float32
        ).astype(jnp.bfloat16)
        bm = jnp.max(logits, axis=1, keepdims=True).astype(jnp.float32)
        m_old = m_s[...]
        m_new = jnp.maximum(m_old, bm)
        p = jnp.exp(logits - m_new.astype(jnp.bfloat16))
        bs = lax.dot_general(
            p, jnp.ones((n, 1), jnp.bfloat16), (((1,), (0,)), ((), ())),
            preferred_element_type=jnp.float32)
        s_s[...] = s_s[...] * jnp.exp(m_old - m_new) + bs
        m_s[...] = m_new

    @pl.when(i < _NBLK)
    def _lut():
        update(lut_ref[...])

    @pl.when(i == _NBLK)
    def _fin():
        update(cq_ref[...])
        o_ref[...] = m_s[...] + jnp.log(s_s[...])


def _stream_lse(x_bf, lut_bf, cq_bf):
    return pl.pallas_call(
        _lse_body,
        grid=(_NBLK + 1,),
        in_specs=[
            pl.BlockSpec((_B, _D), lambda i: (0, 0)),
            pl.BlockSpec((_BN, _D), lambda i: (jnp.minimum(i, _NBLK - 1), 0)),
            pl.BlockSpec((_NUM_CQ, _D), lambda i: (0, 0)),
        ],
        out_specs=pl.BlockSpec((_B, 1), lambda i: (0, 0)),
        out_shape=jax.ShapeDtypeStruct((_B, 1), jnp.float32),
        scratch_shapes=[
            pltpu.VMEM((_B, 1), jnp.float32),
            pltpu.VMEM((_B, 1), jnp.float32),
        ],
    )(x_bf, lut_bf, cq_bf)


def _combine_body(x_ref, first_ref, second_ref, mid_ref, fidx_ref,
                  labc_ref, lab2c_ref, lab2r_ref, w_ref, b_ref,
                  lse_ref, tgt_ref,
                  oim_ref, trip_ref, cov_ref,
                  feat_s, sqr_s, accl_s, accn_s):
    i = pl.program_id(0)

    @pl.when(i == 0)
    def _setup():
        lab = labc_ref[...]
        keep = lab >= 0
        wid = keep & (lab != 20000)
        bfh = wid & (fidx_ref[...] <= mid_ref[...])

        rj = jnp.maximum(lab, 0) & 7
        def extract(t3_ref):
            acc = jnp.where(rj == 0, t3_ref[:, 0, :], 0.0)
            for r in range(1, 8):
                acc = acc + jnp.where(rj == r, t3_ref[:, r, :], 0.0)
            return acc

        proto = jnp.where(bfh, extract(first_ref), extract(second_ref))
        feat_s[0:_B, :] = x_ref[...]
        feat_s[_B:2 * _B, :] = proto
        f = feat_s[...]
        sqr_s[...] = lax.dot_general(
            jnp.ones((1, _D), jnp.float32), f * f, (((1,), (1,)), ((), ())),
            preferred_element_type=jnp.float32)
        accl_s[0, 0] = 0.0
        accn_s[0, 0] = 0.0

        tdot = jnp.sum(x_ref[...] * extract(tgt_ref), axis=1,
                       keepdims=True) * _SCALAR
        nll = jnp.where(wid, lse_ref[...] - tdot, 0.0)
        cnt = jnp.sum(wid.astype(jnp.float32), keepdims=True)
        oim_ref[...] = jnp.sum(nll, keepdims=True) / jnp.maximum(cnt, 1.0)

        xw = x_ref[...] * w_ref[...] + b_ref[...]
        mn = jnp.min(xw, axis=1, keepdims=True)
        xw = xw - mn
        mx = jnp.max(xw, axis=1, keepdims=True)
        xw = xw / mx * (math.pi / 2.0)
        z = math.sqrt(2.0) * (jnp.cos(xw) + jnp.sin(xw))
        zm = jnp.where(keep, z, 0.0)
        nk = jnp.sum(keep.astype(jnp.float32))
        c = lax.dot_general(zm, zm, (((0,), (0,)), ((), ())),
                            preferred_element_type=jnp.float32) / nk
        e = jnp.sum(zm, axis=0, keepdims=True) / nk
        outer = lax.dot_general(e, e, (((0,), (0,)), ((), ())),
                                preferred_element_type=jnp.float32)
        cm = c - outer
        cm2 = cm * cm
        di = (lax.broadcasted_iota(jnp.int32, (_D, _D), 0)
              == lax.broadcasted_iota(jnp.int32, (_D, _D), 1))
        cov_ref[...] = jnp.sum(jnp.where(di, 0.0, cm2), keepdims=True)

    r0 = i * _RB
    fb = feat_s[pl.ds(r0, _RB), :]
    g = lax.dot_general(fb, feat_s[...], (((1,), (1,)), ((), ())),
                        preferred_element_type=jnp.float32)
    sqc = jnp.sum(fb * fb, axis=1, keepdims=True)
    d2 = sqc + sqr_s[...] - 2.0 * g
    dist = jnp.sqrt(jnp.clip(d2, 1e-12, None))
    labc = lab2c_ref[...]
    labr = lab2r_ref[...]
    same = labc == labr
    rvec = lax.broadcasted_iota(jnp.int32, (_RB, 1), 0) + r0
    cvec = lax.broadcasted_iota(jnp.int32, (1, 2 * _B), 1)
    eye = rvec == cvec
    vc = (labc >= 0) & ((rvec < _B) | (labc != 20000))
    vr = (labr >= 0) & ((cvec < _B) | (labr != 20000))
    pair_ok = vc & vr
    pos = same & (~eye) & pair_ok
    neg = (~same) & pair_ok
    hp = jnp.max(jnp.where(pos, dist, -1e9), axis=1, keepdims=True)
    hn = jnp.min(jnp.where(neg, dist, 1e9), axis=1, keepdims=True)
    anyp = jnp.any(pos, axis=1, keepdims=True)
    anyn = jnp.any(neg, axis=1, keepdims=True)
    validr = anyp & anyn
    l = jnp.maximum(hp - hn + _MARGIN, 0.0)
    l = jnp.where(validr, l, 0.0)
    accl_s[0, 0] += jnp.sum(l)
    accn_s[0, 0] += jnp.sum(validr.astype(jnp.float32))

    @pl.when(i == _NRB - 1)
    def _fin():
        val = accl_s[0, 0] / jnp.maximum(accn_s[0, 0], 1.0)
        trip_ref[...] = jnp.full((1, 1), val, jnp.float32)


def _combine_losses(x, first_rows, second_rows, mid_col, fidx_col,
                    lab_col, lab2_col, lab2_row, rff_w, b_row,
                    lse, tgt_rows):
    full = lambda s: pl.BlockSpec(s, lambda i, n=len(s): (0,) * n)
    return pl.pallas_call(
        _combine_body,
        grid=(_NRB,),
        in_specs=[
            full((_B, _D)),
            full((_B, 8, _D)),
            full((_B, 8, _D)),
            full((_B, 1)),
            full((_B, 1)),
            full((_B, 1)),
            pl.BlockSpec((_RB, 1), lambda i: (i, 0)),
            full((1, 2 * _B)),
            full((1, 1)),
            full((1, _D)),
            full((_B, 1)),
            full((_B, 8, _D)),
        ],
        out_specs=[full((1, 1)), full((1, 1)), full((1, 1))],
        out_shape=[
            jax.ShapeDtypeStruct((1, 1), jnp.float32),
            jax.ShapeDtypeStruct((1, 1), jnp.float32),
            jax.ShapeDtypeStruct((1, 1), jnp.float32),
        ],
        scratch_shapes=[
            pltpu.VMEM((2 * _B, _D), jnp.float32),
            pltpu.VMEM((1, 2 * _B), jnp.float32),
            pltpu.SMEM((1, 1), jnp.float32),
            pltpu.SMEM((1, 1), jnp.float32),
        ],
    )(x, first_rows, second_rows, mid_col, fidx_col,
      lab_col, lab2_col, lab2_row, rff_w, b_row, lse, tgt_rows)


def kernel(inputs, roi_label, frame_idxes, lut, cq, first_pos_sample,
           second_pos_sample, middle_of_frame_idx, rff_w, rff_b):
    label_all = roi_label - 1
    fidx = jnp.repeat(frame_idxes, _B // frame_idxes.shape[0])

    x_bf = (inputs * _SCALAR).astype(jnp.bfloat16)
    lut_bf = lut.astype(jnp.bfloat16)
    cq_bf = cq.astype(jnp.bfloat16)
    lse = _stream_lse(x_bf, lut_bf, cq_bf)

    lut3 = lut.reshape(_NUM_PIDS // 8, 8, _D)
    first3 = first_pos_sample.reshape(_NUM_PIDS // 8, 8, _D)
    second3 = second_pos_sample.reshape(_NUM_PIDS // 8, 8, _D)
    tgt_rows, first_rows, second_rows, mid = _sc_gather(
        roi_label, lut3, first3, second3, middle_of_frame_idx)

    lab_col = label_all.reshape(_B, 1)
    lab2 = jnp.concatenate([label_all, label_all])
    lab2_col = lab2.reshape(2 * _B, 1)
    lab2_row = lab2.reshape(1, 2 * _B)
    fidx_col = fidx.reshape(_B, 1)
    mid_col = mid.reshape(_B, 1)
    b_row = rff_b.reshape(1, _D)

    oim, trip, cov = _combine_losses(
        inputs, first_rows, second_rows, mid_col, fidx_col,
        lab_col, lab2_col, lab2_row, rff_w, b_row, lse, tgt_rows)
    return jnp.stack([oim[0, 0], cov[0, 0], trip[0, 0]])
